# baseline (speedup 1.0000x reference)
import functools

import jax
import jax.numpy as jnp
from jax import lax
from jax.experimental import pallas as pl
from jax.experimental.pallas import tpu as pltpu
from jax.experimental.pallas import tpu_sc as plsc

_NUM_CORES = 2
_NUM_SUBCORES = 16
_NUM_WORKERS = _NUM_CORES * _NUM_SUBCORES

_WINDOW = 512


def kernel(x, table):
    batch, fields = x.shape
    n = batch * fields
    dim = table.shape[1]
    idx_flat = x.reshape(n)

    b_per_w = n // _NUM_WORKERS
    n_chunks = b_per_w // _WINDOW

    mesh = plsc.VectorSubcoreMesh(core_axis_name="c", subcore_axis_name="s")

    @functools.partial(
        pl.kernel,
        mesh=mesh,
        compiler_params=pltpu.CompilerParams(use_tc_tiling_on_sc=False),
        out_type=jax.ShapeDtypeStruct((n, dim), table.dtype),
        scratch_types=[
            pltpu.VMEM((b_per_w,), jnp.int32),
            pltpu.VMEM((_WINDOW, dim), jnp.float32),
            pltpu.SemaphoreType.DMA,
        ],
    )
    def emb(table_hbm, idx_hbm, out_hbm, idx_v, rows_v, sem):
        wid = lax.axis_index("s") * _NUM_CORES + lax.axis_index("c")
        base = wid * b_per_w
        pltpu.sync_copy(idx_hbm.at[pl.ds(base, b_per_w)], idx_v)

        @pl.loop(0, n_chunks)
        def _(c):
            off = c * _WINDOW
            pltpu.async_copy(
                table_hbm.at[idx_v.at[pl.ds(off, _WINDOW)]], rows_v, sem
            ).wait()
            pltpu.sync_copy(rows_v, out_hbm.at[pl.ds(base + off, _WINDOW)])

    out = emb(table, idx_flat)
    return out.reshape(batch, fields, dim)


# speedup vs baseline: 2.4445x; 2.4445x over previous
import functools

import jax
import jax.numpy as jnp
from jax import lax
from jax.experimental import pallas as pl
from jax.experimental.pallas import tpu as pltpu
from jax.experimental.pallas import tpu_sc as plsc

_NUM_CORES = 2
_NUM_SUBCORES = 16
_NUM_WORKERS = _NUM_CORES * _NUM_SUBCORES

_WINDOW = 512

_VB = 32768
_VBS = 15

_W2 = 8192
_W2S = 13


def _t1_body(in_ref, out_ref):
    y = in_ref[...].T
    h = _VB // 2
    out_ref[...] = jnp.concatenate([y[:h], y[h:]], axis=1)


def _t2_body(in_ref, out_ref):
    x = in_ref[0]
    out_ref[0] = jnp.concatenate([x[:, :64].T, x[:, 64:].T], axis=1)


def kernel(x, table):
    batch, fields = x.shape
    n = batch * fields
    vocab, dim = table.shape
    nblk = (vocab + _VB - 1) // _VB
    vpad = nblk * _VB
    njb = batch // (2 * _W2)

    xt = x.T
    half = _VB // 2
    sig = ((xt >> _VBS) << _VBS) + 2 * (xt & (half - 1)) + ((xt >> (_VBS - 1)) & 1)
    idx_flat = sig.reshape(n)

    t2tab = pl.pallas_call(
        _t1_body,
        grid=(nblk,),
        in_specs=[pl.BlockSpec((dim, _VB), lambda i: (0, i))],
        out_specs=pl.BlockSpec((_VB // 2, 2 * dim), lambda i: (i, 0)),
        out_shape=jax.ShapeDtypeStruct((vpad // 2, 2 * dim), table.dtype),
    )(tt := table.T)
    tflat = t2tab.reshape(vpad, dim)

    b_per_w = n // _NUM_WORKERS
    n_chunks = b_per_w // _WINDOW

    mesh = plsc.VectorSubcoreMesh(core_axis_name="c", subcore_axis_name="s")

    @functools.partial(
        pl.kernel,
        mesh=mesh,
        compiler_params=pltpu.CompilerParams(use_tc_tiling_on_sc=False),
        out_type=jax.ShapeDtypeStruct((n // 2, 2, dim), table.dtype),
        scratch_types=[
            pltpu.VMEM((b_per_w,), jnp.int32),
            pltpu.VMEM((_WINDOW, dim), jnp.float32),
            pltpu.VMEM((_WINDOW, dim), jnp.float32),
            pltpu.SemaphoreType.DMA,
            pltpu.SemaphoreType.DMA,
        ],
    )
    def emb(table_hbm, idx_hbm, out_hbm, idx_v, rows_a, rows_b, sem_a, sem_b):
        wid = lax.axis_index("s") * _NUM_CORES + lax.axis_index("c")
        base = wid * b_per_w
        pltpu.sync_copy(idx_hbm.at[pl.ds(base, b_per_w)], idx_v)

        def start_gather(c, buf, sem):
            pltpu.async_copy(
                table_hbm.at[idx_v.at[pl.ds(c * _WINDOW, _WINDOW)]], buf, sem
            )

        def wait_gather(c, buf, sem):
            pltpu.make_async_copy(
                table_hbm.at[idx_v.at[pl.ds(c * _WINDOW, _WINDOW)]], buf, sem
            ).wait()

        def write_rows(c, buf):
            p0 = base + c * _WINDOW
            f = p0 >> 14
            rem = p0 - (f << 14)
            j = rem >> (_W2S + 1)
            rem2 = rem - (j << (_W2S + 1))
            h = rem2 >> _W2S
            t0 = rem2 - (h << _W2S)
            p2 = (f << 13) + (j << _W2S) + t0
            pltpu.sync_copy(buf, out_hbm.at[pl.ds(p2, _WINDOW), h])

        start_gather(0, rows_a, sem_a)

        @pl.loop(0, n_chunks // 2)
        def _(k):
            c0 = 2 * k

            wait_gather(c0, rows_a, sem_a)
            start_gather(c0 + 1, rows_b, sem_b)
            write_rows(c0, rows_a)

            wait_gather(c0 + 1, rows_b, sem_b)

            @pl.when(k + 1 < n_chunks // 2)
            def _():
                start_gather(c0 + 2, rows_a, sem_a)

            write_rows(c0 + 1, rows_b)

    out = emb(tflat, idx_flat)

    e2 = out.reshape(fields, batch // 2, 2 * dim)
    q = pl.pallas_call(
        _t2_body,
        grid=(fields, njb),
        in_specs=[pl.BlockSpec((1, _W2, 2 * dim), lambda f, j: (f, j, 0))],
        out_specs=pl.BlockSpec((1, dim, 2 * _W2), lambda f, j: (f, 0, j)),
        out_shape=jax.ShapeDtypeStruct((fields, dim, batch), table.dtype),
    )(e2)

    return jnp.transpose(q, (2, 0, 1))
